# baseline (speedup 1.0000x reference)
import functools

import jax
import jax.numpy as jnp
import numpy as np
from jax import lax
from jax.experimental import pallas as pl
from jax.experimental.pallas import tpu as pltpu
from jax.experimental.pallas import tpu_sc as plsc

SEQ = 200
EMB = 32
BATCH = 4096
VOCAB = 1000000
NW = 32

DBLK = 256
NBLK_FULL = VOCAB // DBLK
DTAIL = VOCAB - NBLK_FULL * DBLK
DSPW = NBLK_FULL // NW
DEXTRA = NBLK_FULL - DSPW * NW

NROWS = BATCH * SEQ
ROWS_PER_W = NROWS // NW
CHUNK = 1600
NCHUNK_B = ROWS_PER_W // CHUNK
SUBG = 100
NSUBG = CHUNK // SUBG
REP = CHUNK // SEQ
SCALE = float(np.sqrt(np.float32(EMB)))

PBLK = 256
PSPW = ROWS_PER_W // PBLK

_mesh = plsc.VectorSubcoreMesh(core_axis_name="c", subcore_axis_name="s")


@functools.partial(
    pl.kernel,
    out_type=jax.ShapeDtypeStruct((VOCAB * EMB,), jnp.float32),
    mesh=_mesh,
    scratch_types=[
        pltpu.VMEM((2, DBLK, EMB), jnp.float32),
        pltpu.VMEM((2, DBLK * EMB), jnp.float32),
        pltpu.VMEM((DTAIL, EMB), jnp.float32),
        pltpu.VMEM((DTAIL * EMB,), jnp.float32),
        pltpu.SemaphoreType.DMA,
        pltpu.SemaphoreType.DMA,
    ],
)
def _sc_detile(ptab_hbm, out_hbm, vin, vcomp, tin, tout, sem_in, sem_out):
    wid = lax.axis_index("s") * 2 + lax.axis_index("c")

    def fire_in(s):
        tok0 = (wid * DSPW + s) * DBLK
        return pltpu.async_copy(
            ptab_hbm.at[pl.ds(pl.multiple_of(tok0, DBLK), DBLK)],
            vin.at[lax.rem(s, 2)], sem_in)

    def fire_out(s):
        tok0 = (wid * DSPW + s) * DBLK
        return pltpu.async_copy(
            vcomp.at[lax.rem(s, 2)],
            out_hbm.at[pl.ds(pl.multiple_of(tok0 * EMB, 128), DBLK * EMB)],
            sem_out)

    def drain_in():
        pltpu.make_async_copy(
            ptab_hbm.at[pl.ds(0, DBLK)], vin.at[0], sem_in).wait()

    def drain_out():
        pltpu.make_async_copy(
            vcomp.at[0], out_hbm.at[pl.ds(0, DBLK * EMB)], sem_out).wait()

    def compact(p, nrows, src, dst):
        def cbody(r, carry):
            dst[p, pl.ds(r * EMB, 16)] = src[p, r, pl.ds(0, 16)]
            dst[p, pl.ds(r * EMB + 16, 16)] = src[p, r, pl.ds(16, 16)]
            return carry

        lax.fori_loop(0, nrows, cbody, 0, unroll=8)

    fire_in(0)

    def step_body(s, carry):
        p = lax.rem(s, 2)

        @pl.when(s + 1 < DSPW)
        def _():
            fire_in(s + 1)

        drain_in()

        @pl.when(s >= 2)
        def _():
            drain_out()

        compact(p, DBLK, vin, vcomp)
        fire_out(s)
        return carry

    lax.fori_loop(0, DSPW, step_body, 0)
    drain_out()
    drain_out()

    @pl.when(wid < DEXTRA)
    def _():
        tok0 = (NBLK_FULL - DEXTRA + wid) * DBLK
        pltpu.sync_copy(
            ptab_hbm.at[pl.ds(pl.multiple_of(tok0, DBLK), DBLK)], vin.at[0])
        compact(0, DBLK, vin, vcomp)
        pltpu.sync_copy(
            vcomp.at[0],
            out_hbm.at[pl.ds(pl.multiple_of(tok0 * EMB, 128), DBLK * EMB)])

    @pl.when(wid == DEXTRA)
    def _():
        tok0 = NBLK_FULL * DBLK
        pltpu.sync_copy(
            ptab_hbm.at[pl.ds(pl.multiple_of(tok0, 64), DTAIL)], tin)

        def tbody(r, carry):
            tout[pl.ds(r * EMB, 16)] = tin[r, pl.ds(0, 16)]
            tout[pl.ds(r * EMB + 16, 16)] = tin[r, pl.ds(16, 16)]
            return carry

        lax.fori_loop(0, DTAIL, tbody, 0, unroll=8)
        pltpu.sync_copy(
            tout, out_hbm.at[pl.ds(pl.multiple_of(tok0 * EMB, 128),
                                   DTAIL * EMB)])


@functools.partial(
    pl.kernel,
    out_type=jax.ShapeDtypeStruct((NROWS, EMB), jnp.float32),
    mesh=_mesh,
    compiler_params=pltpu.CompilerParams(use_tc_tiling_on_sc=False),
    scratch_types=[
        pltpu.VMEM((NSUBG, SUBG), jnp.int32),
        pltpu.VMEM((CHUNK, EMB), jnp.float32),
        pltpu.VMEM((SEQ, EMB), jnp.float32),
        pltpu.SemaphoreType.DMA,
    ],
)
def _sc_embed(idx_hbm, table_hbm, pos_hbm, out_hbm, idx_v, rows_v, pos_v, sem):
    wid = lax.axis_index("s") * 2 + lax.axis_index("c")
    pltpu.sync_copy(pos_hbm, pos_v)

    def chunk_body(c, carry):
        r0 = (wid * NCHUNK_B + c) * NSUBG
        rb = (wid * NCHUNK_B + c) * CHUNK
        pltpu.sync_copy(idx_hbm.at[pl.ds(r0, NSUBG)], idx_v)
        copies = []
        for j in range(NSUBG):
            copies.append(
                pltpu.async_copy(
                    table_hbm.at[idx_v.at[j]],
                    rows_v.at[pl.ds(j * SUBG, SUBG)],
                    sem,
                )
            )
        for cp in copies:
            cp.wait()

        def pos_body(s, carry2):
            p0 = pos_v[s, pl.ds(0, 16)]
            p1 = pos_v[s, pl.ds(16, 16)]
            for k in range(REP):
                r = s + SEQ * k
                rows_v[r, pl.ds(0, 16)] = rows_v[r, pl.ds(0, 16)] * SCALE + p0
                rows_v[r, pl.ds(16, 16)] = rows_v[r, pl.ds(16, 16)] * SCALE + p1
            return carry2

        lax.fori_loop(0, SEQ, pos_body, 0)
        pltpu.sync_copy(rows_v, out_hbm.at[pl.ds(rb, CHUNK)])
        return carry

    lax.fori_loop(0, NCHUNK_B, chunk_body, 0)


@functools.partial(
    pl.kernel,
    out_type=jax.ShapeDtypeStruct((NROWS, EMB), jnp.float32),
    mesh=_mesh,
    scratch_types=[
        pltpu.VMEM((2, PBLK * EMB // 128, 128), jnp.float32),
        pltpu.VMEM((2, PBLK, EMB), jnp.float32),
        pltpu.SemaphoreType.DMA,
        pltpu.SemaphoreType.DMA,
    ],
)
def _sc_pad(lin_hbm, out_hbm, vin, vpad, sem_in, sem_out):
    wid = lax.axis_index("s") * 2 + lax.axis_index("c")
    inr = PBLK * EMB // 128

    def fire_in(s):
        row0 = (wid * PSPW + s) * inr
        return pltpu.async_copy(
            lin_hbm.at[pl.ds(pl.multiple_of(row0, inr), inr)],
            vin.at[lax.rem(s, 2)], sem_in)

    def fire_out(s):
        vrow0 = (wid * PSPW + s) * PBLK
        return pltpu.async_copy(
            vpad.at[lax.rem(s, 2)],
            out_hbm.at[pl.ds(pl.multiple_of(vrow0, PBLK), PBLK)], sem_out)

    def drain_in():
        pltpu.make_async_copy(
            lin_hbm.at[pl.ds(0, inr)], vin.at[0], sem_in).wait()

    def drain_out():
        pltpu.make_async_copy(
            vpad.at[0], out_hbm.at[pl.ds(0, PBLK)], sem_out).wait()

    fire_in(0)

    def step_body(s, carry):
        p = lax.rem(s, 2)

        @pl.when(s + 1 < PSPW)
        def _():
            fire_in(s + 1)

        drain_in()

        @pl.when(s >= 2)
        def _():
            drain_out()

        def cbody(r, carry2):
            q = r >> 2
            o = (r & 3) * EMB
            vpad[p, r, pl.ds(0, 16)] = vin[p, q, pl.ds(o, 16)]
            vpad[p, r, pl.ds(16, 16)] = vin[p, q, pl.ds(o + 16, 16)]
            return carry2

        lax.fori_loop(0, PBLK, cbody, 0, unroll=8)
        fire_out(s)
        return carry

    lax.fori_loop(0, PSPW, step_body, 0)
    drain_out()
    drain_out()


def kernel(inputs, token_table, pos_table):
    table_lin = _sc_detile(token_table)
    idx = inputs.reshape(-1).astype(jnp.int32).reshape(NROWS // SUBG, SUBG)
    out_lin = _sc_embed(idx, table_lin.reshape(VOCAB, EMB), pos_table)
    out = _sc_pad(out_lin.reshape(NROWS * EMB // 128, 128))
    return out.reshape(BATCH, SEQ, EMB)


# speedup vs baseline: 1.0667x; 1.0667x over previous
import functools

import jax
import jax.numpy as jnp
import numpy as np
from jax import lax
from jax.experimental import pallas as pl
from jax.experimental.pallas import tpu as pltpu
from jax.experimental.pallas import tpu_sc as plsc

SEQ = 200
EMB = 32
BATCH = 4096
VOCAB = 1000000
NW = 32

DBLK = 256
NBLK_FULL = VOCAB // DBLK
DTAIL = VOCAB - NBLK_FULL * DBLK
DSPW = NBLK_FULL // NW
DEXTRA = NBLK_FULL - DSPW * NW

NROWS = BATCH * SEQ
ROWS_PER_W = NROWS // NW
CHUNK = 1600
NCHUNK_B = ROWS_PER_W // CHUNK
SUBG = 100
NSUBG = CHUNK // SUBG
REP = CHUNK // SEQ
SCALE = float(np.sqrt(np.float32(EMB)))

PBLK = 256
PSPW = ROWS_PER_W // PBLK

_mesh = plsc.VectorSubcoreMesh(core_axis_name="c", subcore_axis_name="s")


@functools.partial(
    pl.kernel,
    out_type=jax.ShapeDtypeStruct((NROWS, EMB), jnp.float32),
    mesh=_mesh,
    compiler_params=pltpu.CompilerParams(use_tc_tiling_on_sc=False),
    scratch_types=[
        pltpu.VMEM((NSUBG, SUBG), jnp.int32),
        pltpu.VMEM((CHUNK, EMB), jnp.float32),
        pltpu.VMEM((SEQ, EMB), jnp.float32),
        pltpu.SemaphoreType.DMA,
    ],
)
def _sc_embed(idx_hbm, table_hbm, pos_hbm, out_hbm, idx_v, rows_v, pos_v, sem):
    wid = lax.axis_index("s") * 2 + lax.axis_index("c")
    pltpu.sync_copy(pos_hbm, pos_v)

    def chunk_body(c, carry):
        r0 = (wid * NCHUNK_B + c) * NSUBG
        rb = (wid * NCHUNK_B + c) * CHUNK
        pltpu.sync_copy(idx_hbm.at[pl.ds(r0, NSUBG)], idx_v)
        copies = []
        for j in range(NSUBG):
            copies.append(
                pltpu.async_copy(
                    table_hbm.at[idx_v.at[j]],
                    rows_v.at[pl.ds(j * SUBG, SUBG)],
                    sem,
                )
            )
        for cp in copies:
            cp.wait()

        def pos_body(s, carry2):
            p0 = pos_v[s, pl.ds(0, 16)]
            p1 = pos_v[s, pl.ds(16, 16)]
            for k in range(REP):
                r = s + SEQ * k
                rows_v[r, pl.ds(0, 16)] = rows_v[r, pl.ds(0, 16)] * SCALE + p0
                rows_v[r, pl.ds(16, 16)] = rows_v[r, pl.ds(16, 16)] * SCALE + p1
            return carry2

        lax.fori_loop(0, SEQ, pos_body, 0)
        pltpu.sync_copy(rows_v, out_hbm.at[pl.ds(rb, CHUNK)])
        return carry

    lax.fori_loop(0, NCHUNK_B, chunk_body, 0)


@functools.partial(
    pl.kernel,
    out_type=jax.ShapeDtypeStruct((NROWS, EMB), jnp.float32),
    mesh=_mesh,
    scratch_types=[
        pltpu.VMEM((2, PBLK * EMB // 128, 128), jnp.float32),
        pltpu.VMEM((2, PBLK, EMB), jnp.float32),
        pltpu.SemaphoreType.DMA,
        pltpu.SemaphoreType.DMA,
    ],
)
def _sc_pad(lin_hbm, out_hbm, vin, vpad, sem_in, sem_out):
    wid = lax.axis_index("s") * 2 + lax.axis_index("c")
    inr = PBLK * EMB // 128

    def fire_in(s):
        row0 = (wid * PSPW + s) * inr
        return pltpu.async_copy(
            lin_hbm.at[pl.ds(pl.multiple_of(row0, inr), inr)],
            vin.at[lax.rem(s, 2)], sem_in)

    def fire_out(s):
        vrow0 = (wid * PSPW + s) * PBLK
        return pltpu.async_copy(
            vpad.at[lax.rem(s, 2)],
            out_hbm.at[pl.ds(pl.multiple_of(vrow0, PBLK), PBLK)], sem_out)

    def drain_in():
        pltpu.make_async_copy(
            lin_hbm.at[pl.ds(0, inr)], vin.at[0], sem_in).wait()

    def drain_out():
        pltpu.make_async_copy(
            vpad.at[0], out_hbm.at[pl.ds(0, PBLK)], sem_out).wait()

    fire_in(0)

    def step_body(s, carry):
        p = lax.rem(s, 2)

        @pl.when(s + 1 < PSPW)
        def _():
            fire_in(s + 1)

        drain_in()

        @pl.when(s >= 2)
        def _():
            drain_out()

        def cbody(r, carry2):
            q = r >> 2
            o = (r & 3) * EMB
            vpad[p, r, pl.ds(0, 16)] = vin[p, q, pl.ds(o, 16)]
            vpad[p, r, pl.ds(16, 16)] = vin[p, q, pl.ds(o + 16, 16)]
            return carry2

        lax.fori_loop(0, PBLK, cbody, 0, unroll=8)
        fire_out(s)
        return carry

    lax.fori_loop(0, PSPW, step_body, 0)
    drain_out()
    drain_out()


def kernel(inputs, token_table, pos_table):
    idx = inputs.reshape(-1).astype(jnp.int32).reshape(NROWS // SUBG, SUBG)
    out_lin = _sc_embed(idx, token_table, pos_table)
    out = _sc_pad(out_lin.reshape(NROWS * EMB // 128, 128))
    return out.reshape(BATCH, SEQ, EMB)


# speedup vs baseline: 1.1121x; 1.0426x over previous
import functools

import jax
import jax.numpy as jnp
import numpy as np
from jax import lax
from jax.experimental import pallas as pl
from jax.experimental.pallas import tpu as pltpu
from jax.experimental.pallas import tpu_sc as plsc

SEQ = 200
EMB = 32
BATCH = 4096
VOCAB = 1000000
NW = 32

DBLK = 256
NBLK_FULL = VOCAB // DBLK
DTAIL = VOCAB - NBLK_FULL * DBLK
DSPW = NBLK_FULL // NW
DEXTRA = NBLK_FULL - DSPW * NW

NROWS = BATCH * SEQ
ROWS_PER_W = NROWS // NW
CHUNK = 1600
NCHUNK_B = ROWS_PER_W // CHUNK
SUBG = 100
NSUBG = CHUNK // SUBG
REP = CHUNK // SEQ
SCALE = float(np.sqrt(np.float32(EMB)))

PBLK = 256
PSPW = ROWS_PER_W // PBLK

_mesh = plsc.VectorSubcoreMesh(core_axis_name="c", subcore_axis_name="s")


@functools.partial(
    pl.kernel,
    out_type=jax.ShapeDtypeStruct((NROWS, EMB), jnp.float32),
    mesh=_mesh,
    compiler_params=pltpu.CompilerParams(use_tc_tiling_on_sc=False),
    scratch_types=[
        pltpu.VMEM((NSUBG, SUBG), jnp.int32),
        pltpu.VMEM((NSUBG, SUBG), jnp.int32),
        pltpu.VMEM((CHUNK, EMB), jnp.float32),
        pltpu.VMEM((CHUNK, EMB), jnp.float32),
        pltpu.VMEM((SEQ, EMB), jnp.float32),
        pltpu.SemaphoreType.DMA,
        pltpu.SemaphoreType.DMA,
    ],
)
def _sc_embed(idx_hbm, table_hbm, pos_hbm, out_hbm,
              idx0, idx1, rows0, rows1, pos_v, sem0, sem1):
    wid = lax.axis_index("s") * 2 + lax.axis_index("c")
    pltpu.sync_copy(pos_hbm, pos_v)
    idxs = (idx0, idx1)
    rows = (rows0, rows1)
    sems = (sem0, sem1)

    def fetch(c, p):
        r0 = (wid * NCHUNK_B + c) * NSUBG
        pltpu.sync_copy(idx_hbm.at[pl.ds(r0, NSUBG)], idxs[p])
        for j in range(NSUBG):
            pltpu.async_copy(
                table_hbm.at[idxs[p].at[j]],
                rows[p].at[pl.ds(j * SUBG, SUBG)],
                sems[p],
            )

    def process(c, p):
        pltpu.make_async_copy(
            table_hbm.at[pl.ds(0, CHUNK)], rows[p], sems[p]).wait()
        rv = rows[p]

        def pos_body(s, carry2):
            p0 = pos_v[s, pl.ds(0, 16)]
            p1 = pos_v[s, pl.ds(16, 16)]
            for k in range(REP):
                r = s + SEQ * k
                rv[r, pl.ds(0, 16)] = rv[r, pl.ds(0, 16)] * SCALE + p0
                rv[r, pl.ds(16, 16)] = rv[r, pl.ds(16, 16)] * SCALE + p1
            return carry2

        lax.fori_loop(0, SEQ, pos_body, 0)
        rb = (wid * NCHUNK_B + c) * CHUNK
        pltpu.sync_copy(rv, out_hbm.at[pl.ds(rb, CHUNK)])

    fetch(0, 0)

    def pair_body(i, carry):
        c0 = i * 2
        fetch(c0 + 1, 1)
        process(c0, 0)

        @pl.when(c0 + 2 < NCHUNK_B)
        def _():
            fetch(c0 + 2, 0)

        process(c0 + 1, 1)
        return carry

    lax.fori_loop(0, NCHUNK_B // 2, pair_body, 0)


@functools.partial(
    pl.kernel,
    out_type=jax.ShapeDtypeStruct((NROWS, EMB), jnp.float32),
    mesh=_mesh,
    scratch_types=[
        pltpu.VMEM((2, PBLK * EMB // 128, 128), jnp.float32),
        pltpu.VMEM((2, PBLK, EMB), jnp.float32),
        pltpu.SemaphoreType.DMA,
        pltpu.SemaphoreType.DMA,
    ],
)
def _sc_pad(lin_hbm, out_hbm, vin, vpad, sem_in, sem_out):
    wid = lax.axis_index("s") * 2 + lax.axis_index("c")
    inr = PBLK * EMB // 128

    def fire_in(s):
        row0 = (wid * PSPW + s) * inr
        return pltpu.async_copy(
            lin_hbm.at[pl.ds(pl.multiple_of(row0, inr), inr)],
            vin.at[lax.rem(s, 2)], sem_in)

    def fire_out(s):
        vrow0 = (wid * PSPW + s) * PBLK
        return pltpu.async_copy(
            vpad.at[lax.rem(s, 2)],
            out_hbm.at[pl.ds(pl.multiple_of(vrow0, PBLK), PBLK)], sem_out)

    def drain_in():
        pltpu.make_async_copy(
            lin_hbm.at[pl.ds(0, inr)], vin.at[0], sem_in).wait()

    def drain_out():
        pltpu.make_async_copy(
            vpad.at[0], out_hbm.at[pl.ds(0, PBLK)], sem_out).wait()

    fire_in(0)

    def step_body(s, carry):
        p = lax.rem(s, 2)

        @pl.when(s + 1 < PSPW)
        def _():
            fire_in(s + 1)

        drain_in()

        @pl.when(s >= 2)
        def _():
            drain_out()

        def cbody(r, carry2):
            q = r >> 2
            o = (r & 3) * EMB
            vpad[p, r, pl.ds(0, 16)] = vin[p, q, pl.ds(o, 16)]
            vpad[p, r, pl.ds(16, 16)] = vin[p, q, pl.ds(o + 16, 16)]
            return carry2

        lax.fori_loop(0, PBLK, cbody, 0, unroll=8)
        fire_out(s)
        return carry

    lax.fori_loop(0, PSPW, step_body, 0)
    drain_out()
    drain_out()


def kernel(inputs, token_table, pos_table):
    idx = inputs.reshape(-1).astype(jnp.int32).reshape(NROWS // SUBG, SUBG)
    out_lin = _sc_embed(idx, token_table, pos_table)
    out = _sc_pad(out_lin.reshape(NROWS * EMB // 128, 128))
    return out.reshape(BATCH, SEQ, EMB)
